# baseline (speedup 1.0000x reference)
import functools
import math

import numpy as np
import jax
import jax.numpy as jnp
from jax.experimental import pallas as pl
from jax.experimental.pallas import tpu as pltpu

_FACTOR = 5
_NEG = -1e30
_U32 = np.uint32

_mask_cache = {}


def _tf_block(key2, x0, x1):
    x0 = x0.astype(_U32).copy()
    x1 = x1.astype(_U32).copy()
    ks0, ks1 = _U32(key2[0]), _U32(key2[1])
    ks2 = _U32(ks0 ^ ks1 ^ _U32(0x1BD11BDA))
    rot = [13, 15, 26, 6, 17, 29, 16, 24]

    def rotl(x, r):
        return ((x << _U32(r)) | (x >> _U32(32 - r))).astype(_U32)

    with np.errstate(over="ignore"):
        x0 += ks0
        x1 += ks1
        ks = [ks1, ks2, ks0]
        for g in range(5):
            for r in rot[(g % 2) * 4:(g % 2) * 4 + 4]:
                x0 = (x0 + x1).astype(_U32)
                x1 = rotl(x1, r)
                x1 = (x1 ^ x0).astype(_U32)
            x0 = (x0 + ks[g % 3]).astype(_U32)
            x1 = (x1 + ks[(g + 1) % 3] + _U32(g + 1)).astype(_U32)
    return x0, x1


def _np_index_sample(L_Q, U_part, L_K):
    n = L_Q * U_part
    key0 = np.zeros(2, _U32)
    f0, f1 = _tf_block(key0, np.zeros(1, _U32), np.array([1234], _U32))
    folded = np.array([f0[0], f1[0]], _U32)
    s0, s1 = _tf_block(folded, np.zeros(2, _U32), np.arange(2, dtype=_U32))
    k1 = np.array([s0[0], s1[0]], _U32)
    k2 = np.array([s0[1], s1[1]], _U32)

    def bits(k):
        b0, b1 = _tf_block(k, np.zeros(n, _U32), np.arange(n, dtype=_U32))
        return (b0 ^ b1).astype(_U32)

    span = _U32(L_K)
    mult = _U32((int(2 ** 16 % span) ** 2) % span)
    with np.errstate(over="ignore"):
        off = (bits(k1) % span) * mult + bits(k2) % span
        off = off % span
    return off.astype(np.int32).reshape(L_Q, U_part)


def _sample_masks(L_Q, L_K, U_part):
    ck = (L_Q, L_K, U_part)
    if ck not in _mask_cache:
        index_sample = _np_index_sample(L_Q, U_part, L_K)
        cnt = np.zeros((L_Q, L_K), np.float32)
        np.add.at(cnt, (np.arange(L_Q)[:, None], index_sample), 1.0)
        bias = np.where(cnt > 0, 0.0, _NEG).astype(np.float32)
        _mask_cache[ck] = (bias, cnt)
    return _mask_cache[ck]


def _body(q_ref, k_ref, v_ref, bias_ref, cnt_ref, out_ref,
          m_ref, idx_ref, qsel_ref, ctx_ref, *, u, scale, tq, D):
    HI = jax.lax.Precision.HIGHEST
    f32 = jnp.float32
    L_Q = q_ref.shape[1]
    L_K = k_ref.shape[1]
    n_t = L_Q // tq
    row_io = jax.lax.broadcasted_iota(jnp.int32, (n_t, tq), 0)
    col_io = jax.lax.broadcasted_iota(jnp.int32, (n_t, tq), 1)
    gidx = row_io * tq + col_io

    out_ref[0] = jnp.zeros(out_ref.shape[1:], f32)

    for sub in range(2):
        lo = sub * D
        kh = k_ref[0, :, lo:lo + D]
        vh = v_ref[0, :, lo:lo + D]

        for t in range(n_t):
            qt = q_ref[0, pl.ds(t * tq, tq), lo:lo + D]
            st = jax.lax.dot_general(qt, kh, (((1,), (1,)), ((), ())),
                                     preferred_element_type=f32)
            bf = bias_ref[pl.ds(t * tq, tq), :]
            cf = cnt_ref[pl.ds(t * tq, tq), :]
            mx = jnp.max(st + bf, axis=1)
            sm = jnp.sum(st * cf, axis=1)
            m_ref[pl.ds(t, 1), :] = (mx - sm * (1.0 / L_K)).reshape(1, tq)

        qsel_ref[:] = jnp.zeros(qsel_ref.shape, f32)

        def topk_body(i, _):
            mv = m_ref[:]
            mmax = jnp.max(mv)
            j = jnp.min(jnp.where(mv == mmax, gidx, L_Q))
            idx_ref[i] = j
            m_ref[:] = jnp.where(gidx == j, _NEG, mv)
            qsel_ref[pl.ds(i, 1), :] = q_ref[0, pl.ds(j, 1), lo:lo + D]
            return 0

        jax.lax.fori_loop(0, u, topk_body, 0)

        qsel = qsel_ref[:]
        ssel = jax.lax.dot_general(qsel, kh, (((1,), (1,)), ((), ())),
                                   precision=HI, preferred_element_type=f32)
        logits = ssel * scale
        p = jnp.exp(logits - jnp.max(logits, axis=1, keepdims=True))
        attn = p / jnp.sum(p, axis=1, keepdims=True)
        ctx_ref[:] = jax.lax.dot_general(attn, vh, (((1,), (0,)), ((), ())),
                                         precision=HI,
                                         preferred_element_type=f32)

        def scat_body(i, _):
            out_ref[0, pl.ds(idx_ref[i], 1), lo:lo + D] = ctx_ref[pl.ds(i, 1), :]
            return 0

        jax.lax.fori_loop(0, u, scat_body, 0)


def kernel(queries, keys, values, attn_mask):
    B, L_Q, H, D = queries.shape
    L_K = keys.shape[1]
    U_part = min(_FACTOR * int(math.ceil(math.log(L_K))), L_K)
    u = min(_FACTOR * int(math.ceil(math.log(L_Q))), L_Q)
    upad = max(8, ((u + 7) // 8) * 8)
    tq = 256
    bias_np, cnt_np = _sample_masks(L_Q, L_K, U_part)

    Q = queries.reshape(B, L_Q, H * D)
    K = keys.reshape(B, L_K, H * D)
    V = values.reshape(B, L_K, H * D)
    n_pair = B * (H // 2)

    body = functools.partial(_body, u=u, scale=1.0 / math.sqrt(D),
                             tq=tq, D=D)
    hp = H // 2
    out = pl.pallas_call(
        body,
        grid=(n_pair,),
        in_specs=[
            pl.BlockSpec((1, L_Q, 2 * D), lambda i: (i // hp, 0, i % hp)),
            pl.BlockSpec((1, L_K, 2 * D), lambda i: (i // hp, 0, i % hp)),
            pl.BlockSpec((1, L_K, 2 * D), lambda i: (i // hp, 0, i % hp)),
            pl.BlockSpec((L_Q, L_K), lambda i: (0, 0)),
            pl.BlockSpec((L_Q, L_K), lambda i: (0, 0)),
        ],
        out_specs=pl.BlockSpec((1, L_Q, 2 * D), lambda i: (i // hp, 0, i % hp)),
        out_shape=jax.ShapeDtypeStruct((B, L_Q, H * D), jnp.float32),
        scratch_shapes=[
            pltpu.VMEM((L_Q // tq, tq), jnp.float32),
            pltpu.SMEM((upad,), jnp.int32),
            pltpu.VMEM((upad, D), jnp.float32),
            pltpu.VMEM((upad, D), jnp.float32),
        ],
        compiler_params=pltpu.CompilerParams(
            dimension_semantics=("arbitrary",)),
    )(Q, K, V, jnp.asarray(bias_np), jnp.asarray(cnt_np))

    return out.reshape(B, L_Q, H, D)


# speedup vs baseline: 1.0357x; 1.0357x over previous
import functools
import math

import numpy as np
import jax
import jax.numpy as jnp
from jax.experimental import pallas as pl
from jax.experimental.pallas import tpu as pltpu

_FACTOR = 5
_NEG = -1e30
_U32 = np.uint32

_mask_cache = {}


def _tf_block(key2, x0, x1):
    x0 = x0.astype(_U32).copy()
    x1 = x1.astype(_U32).copy()
    ks0, ks1 = _U32(key2[0]), _U32(key2[1])
    ks2 = _U32(ks0 ^ ks1 ^ _U32(0x1BD11BDA))
    rot = [13, 15, 26, 6, 17, 29, 16, 24]

    def rotl(x, r):
        return ((x << _U32(r)) | (x >> _U32(32 - r))).astype(_U32)

    with np.errstate(over="ignore"):
        x0 += ks0
        x1 += ks1
        ks = [ks1, ks2, ks0]
        for g in range(5):
            for r in rot[(g % 2) * 4:(g % 2) * 4 + 4]:
                x0 = (x0 + x1).astype(_U32)
                x1 = rotl(x1, r)
                x1 = (x1 ^ x0).astype(_U32)
            x0 = (x0 + ks[g % 3]).astype(_U32)
            x1 = (x1 + ks[(g + 1) % 3] + _U32(g + 1)).astype(_U32)
    return x0, x1


def _np_index_sample(L_Q, U_part, L_K):
    n = L_Q * U_part
    key0 = np.zeros(2, _U32)
    f0, f1 = _tf_block(key0, np.zeros(1, _U32), np.array([1234], _U32))
    folded = np.array([f0[0], f1[0]], _U32)
    s0, s1 = _tf_block(folded, np.zeros(2, _U32), np.arange(2, dtype=_U32))
    k1 = np.array([s0[0], s1[0]], _U32)
    k2 = np.array([s0[1], s1[1]], _U32)

    def bits(k):
        b0, b1 = _tf_block(k, np.zeros(n, _U32), np.arange(n, dtype=_U32))
        return (b0 ^ b1).astype(_U32)

    span = _U32(L_K)
    mult = _U32((int(2 ** 16 % span) ** 2) % span)
    with np.errstate(over="ignore"):
        off = (bits(k1) % span) * mult + bits(k2) % span
        off = off % span
    return off.astype(np.int32).reshape(L_Q, U_part)


def _sample_masks(L_Q, L_K, U_part):
    ck = (L_Q, L_K, U_part)
    if ck not in _mask_cache:
        index_sample = _np_index_sample(L_Q, U_part, L_K)
        cnt = np.zeros((L_Q, L_K), np.float32)
        np.add.at(cnt, (np.arange(L_Q)[:, None], index_sample), 1.0)
        bias = np.where(cnt > 0, 0.0, _NEG).astype(np.float32)
        _mask_cache[ck] = (bias, cnt)
    return _mask_cache[ck]


def _body(q_ref, k_ref, v_ref, bias_hbm, cnt_hbm, out_ref,
          m_ref, idxrow_ref, qsel_ref, bias_ref, cnt_ref, dma_sem,
          *, u, upad, scale, tq, D):
    HI = jax.lax.Precision.HIGHEST
    f32 = jnp.float32
    L_Q = q_ref.shape[1]
    L_K = k_ref.shape[1]
    n_t = L_Q // tq

    @pl.when(pl.program_id(0) == 0)
    def _load_masks():
        c1 = pltpu.make_async_copy(bias_hbm, bias_ref, dma_sem)
        c1.start()
        c1.wait()
        c2 = pltpu.make_async_copy(cnt_hbm, cnt_ref, dma_sem)
        c2.start()
        c2.wait()

    row_io = jax.lax.broadcasted_iota(jnp.int32, (n_t, tq), 0)
    col_io = jax.lax.broadcasted_iota(jnp.int32, (n_t, tq), 1)
    gidx = row_io * tq + col_io
    lane_u = jax.lax.broadcasted_iota(jnp.int32, (1, upad), 1)
    sub_k = jax.lax.broadcasted_iota(jnp.int32, (L_Q, upad), 0)
    lane_k = jax.lax.broadcasted_iota(jnp.int32, (upad, L_K), 1)

    for sub in range(2):
        lo = sub * D
        kh = k_ref[0, :, lo:lo + D]
        vh = v_ref[0, :, lo:lo + D]

        for t in range(n_t):
            qt = q_ref[0, pl.ds(t * tq, tq), lo:lo + D]
            st = jax.lax.dot_general(qt, kh, (((1,), (1,)), ((), ())),
                                     preferred_element_type=f32)
            bf = bias_ref[pl.ds(t * tq, tq), :]
            cf = cnt_ref[pl.ds(t * tq, tq), :]
            mx = jnp.max(st + bf, axis=1)
            sm = jnp.sum(st * cf, axis=1)
            m_ref[pl.ds(t, 1), :] = (mx - sm * (1.0 / L_K)).reshape(1, tq)

        idxrow_ref[:] = jnp.zeros((1, upad), jnp.int32) + L_Q

        def topk_body(i, _):
            mv = m_ref[:]
            mmax = jnp.max(mv)
            j = jnp.min(jnp.where(mv == mmax, gidx, L_Q))
            idxrow_ref[:] = jnp.where(lane_u == i, j, idxrow_ref[:])
            m_ref[:] = jnp.where(gidx == j, _NEG, mv)
            return 0

        jax.lax.fori_loop(0, u, topk_body, 0)

        idxrow = idxrow_ref[:]
        pt = (sub_k == idxrow).astype(f32)
        idxcol = jnp.transpose(idxrow, (1, 0))
        pg = (lane_k == idxcol).astype(f32)
        qh = q_ref[0, :, lo:lo + D]
        qsel_ref[:] = jax.lax.dot_general(
            pg, qh, (((1,), (0,)), ((), ())),
            precision=HI, preferred_element_type=f32)

        qsel = qsel_ref[:]
        ssel = jax.lax.dot_general(qsel, kh, (((1,), (1,)), ((), ())),
                                   precision=HI, preferred_element_type=f32)
        logits = ssel * scale
        p = jnp.exp(logits - jnp.max(logits, axis=1, keepdims=True))
        attn = p / jnp.sum(p, axis=1, keepdims=True)
        ctx = jax.lax.dot_general(attn, vh, (((1,), (0,)), ((), ())),
                                  precision=HI, preferred_element_type=f32)

        out_ref[0, :, lo:lo + D] = jax.lax.dot_general(
            pt, ctx, (((1,), (0,)), ((), ())),
            precision=HI, preferred_element_type=f32)


def kernel(queries, keys, values, attn_mask):
    B, L_Q, H, D = queries.shape
    L_K = keys.shape[1]
    U_part = min(_FACTOR * int(math.ceil(math.log(L_K))), L_K)
    u = min(_FACTOR * int(math.ceil(math.log(L_Q))), L_Q)
    upad = max(8, ((u + 7) // 8) * 8)
    tq = 256
    bias_np, cnt_np = _sample_masks(L_Q, L_K, U_part)

    Q = queries.reshape(B, L_Q, H * D)
    K = keys.reshape(B, L_K, H * D)
    V = values.reshape(B, L_K, H * D)
    hp = H // 2
    n_pair = B * hp

    body = functools.partial(_body, u=u, upad=upad,
                             scale=1.0 / math.sqrt(D), tq=tq, D=D)
    out = pl.pallas_call(
        body,
        grid=(n_pair,),
        in_specs=[
            pl.BlockSpec((1, L_Q, 2 * D), lambda i: (i // hp, 0, i % hp)),
            pl.BlockSpec((1, L_K, 2 * D), lambda i: (i // hp, 0, i % hp)),
            pl.BlockSpec((1, L_K, 2 * D), lambda i: (i // hp, 0, i % hp)),
            pl.BlockSpec(memory_space=pl.ANY),
            pl.BlockSpec(memory_space=pl.ANY),
        ],
        out_specs=pl.BlockSpec((1, L_Q, 2 * D), lambda i: (i // hp, 0, i % hp)),
        out_shape=jax.ShapeDtypeStruct((B, L_Q, H * D), jnp.float32),
        scratch_shapes=[
            pltpu.VMEM((L_Q // tq, tq), jnp.float32),
            pltpu.VMEM((1, upad), jnp.int32),
            pltpu.VMEM((upad, D), jnp.float32),
            pltpu.VMEM((L_Q, L_K), jnp.float32),
            pltpu.VMEM((L_Q, L_K), jnp.float32),
            pltpu.SemaphoreType.DMA,
        ],
        compiler_params=pltpu.CompilerParams(
            dimension_semantics=("arbitrary",)),
    )(Q, K, V, jnp.asarray(bias_np), jnp.asarray(cnt_np))

    return out.reshape(B, L_Q, H, D)


# speedup vs baseline: 2.1574x; 2.0829x over previous
import functools
import math

import numpy as np
import jax
import jax.numpy as jnp
from jax import lax
from jax.experimental import pallas as pl
from jax.experimental.pallas import tpu as pltpu
from jax.experimental.pallas import tpu_sc as plsc

_FACTOR = 5
_NEG = -1e30
_U32 = np.uint32

_mask_cache = {}


def _tf_block(key2, x0, x1):
    x0 = x0.astype(_U32).copy()
    x1 = x1.astype(_U32).copy()
    ks0, ks1 = _U32(key2[0]), _U32(key2[1])
    ks2 = _U32(ks0 ^ ks1 ^ _U32(0x1BD11BDA))
    rot = [13, 15, 26, 6, 17, 29, 16, 24]

    def rotl(x, r):
        return ((x << _U32(r)) | (x >> _U32(32 - r))).astype(_U32)

    with np.errstate(over="ignore"):
        x0 += ks0
        x1 += ks1
        ks = [ks1, ks2, ks0]
        for g in range(5):
            for r in rot[(g % 2) * 4:(g % 2) * 4 + 4]:
                x0 = (x0 + x1).astype(_U32)
                x1 = rotl(x1, r)
                x1 = (x1 ^ x0).astype(_U32)
            x0 = (x0 + ks[g % 3]).astype(_U32)
            x1 = (x1 + ks[(g + 1) % 3] + _U32(g + 1)).astype(_U32)
    return x0, x1


def _np_index_sample(L_Q, U_part, L_K):
    n = L_Q * U_part
    key0 = np.zeros(2, _U32)
    f0, f1 = _tf_block(key0, np.zeros(1, _U32), np.array([1234], _U32))
    folded = np.array([f0[0], f1[0]], _U32)
    s0, s1 = _tf_block(folded, np.zeros(2, _U32), np.arange(2, dtype=_U32))
    k1 = np.array([s0[0], s1[0]], _U32)
    k2 = np.array([s0[1], s1[1]], _U32)

    def bits(k):
        b0, b1 = _tf_block(k, np.zeros(n, _U32), np.arange(n, dtype=_U32))
        return (b0 ^ b1).astype(_U32)

    span = _U32(L_K)
    mult = _U32((int(2 ** 16 % span) ** 2) % span)
    with np.errstate(over="ignore"):
        off = (bits(k1) % span) * mult + bits(k2) % span
        off = off % span
    return off.astype(np.int32).reshape(L_Q, U_part)


def _sample_masks(L_Q, L_K, U_part):
    ck = (L_Q, L_K, U_part)
    if ck not in _mask_cache:
        index_sample = _np_index_sample(L_Q, U_part, L_K)
        cnt = np.zeros((L_Q, L_K), np.float32)
        np.add.at(cnt, (np.arange(L_Q)[:, None], index_sample), 1.0)
        bias = np.where(cnt > 0, 0.0, _NEG).astype(np.float32)
        _mask_cache[ck] = (bias, cnt)
    return _mask_cache[ck]


def _m_body(q_ref, k_ref, bias_hbm, cnt_hbm, m_out,
            bias_ref, cnt_ref, dma_sem, *, tq, D):
    f32 = jnp.float32
    L_Q = q_ref.shape[1]
    L_K = k_ref.shape[1]
    n_t = L_Q // tq

    @pl.when(pl.program_id(0) == 0)
    def _load_masks():
        c1 = pltpu.make_async_copy(bias_hbm, bias_ref, dma_sem)
        c1.start()
        c1.wait()
        c2 = pltpu.make_async_copy(cnt_hbm, cnt_ref, dma_sem)
        c2.start()
        c2.wait()

    for sub in range(2):
        lo = sub * D
        kh = k_ref[0, :, lo:lo + D]
        for t in range(n_t):
            qt = q_ref[0, pl.ds(t * tq, tq), lo:lo + D]
            st = jax.lax.dot_general(qt, kh, (((1,), (1,)), ((), ())),
                                     preferred_element_type=f32)
            bf = bias_ref[pl.ds(t * tq, tq), :]
            cf = cnt_ref[pl.ds(t * tq, tq), :]
            mx = jnp.max(st + bf, axis=1)
            sm = jnp.sum(st * cf, axis=1)
            m_out[0, sub, pl.ds(t * tq, tq)] = mx - sm * (1.0 / L_K)


def _make_sc_topk(n_rows, L, u, upad):
    f32 = jnp.float32
    i32 = jnp.int32
    n_sl = L // 16
    mesh = plsc.VectorSubcoreMesh(core_axis_name="c", subcore_axis_name="s")

    @functools.partial(
        pl.kernel, mesh=mesh,
        out_type=jax.ShapeDtypeStruct((n_rows, upad), i32),
        scratch_types=[
            pltpu.VMEM((L,), f32),
            pltpu.VMEM((upad,), i32),
        ],
        compiler_params=pltpu.CompilerParams(needs_layout_passes=False),
    )
    def sc_topk(m_hbm, idx_hbm, m_v, out_v):
        wid = lax.axis_index("s") * 2 + lax.axis_index("c")
        pltpu.sync_copy(m_hbm.at[wid], m_v)
        lane = lax.broadcasted_iota(i32, (16,), 0)
        neg = jnp.full((16,), -3.0e38, f32)

        def extract(it, acc):
            def scan(i, carry):
                vm, vi = carry
                v = m_v[pl.ds(i * 16, 16)]
                upd = v > vm
                vi = jnp.where(upd, lane + i * 16, vi)
                vm = jnp.where(upd, v, vm)
                return vm, vi

            vm0 = jnp.full((16,), -3.0e38, f32)
            vi0 = jnp.full((16,), L, i32)
            vm, vi = lax.fori_loop(0, n_sl, scan, (vm0, vi0))
            gmax = jnp.max(vm)
            j = jnp.min(jnp.where(vm == gmax, vi, L))
            plsc.store_scatter(m_v, [jnp.full((16,), j, i32)], neg,
                               mask=lane == 0)
            return tuple(
                jnp.where(lane + b * 16 == it, j, acc[b])
                for b in range(upad // 16))

        acc0 = tuple(jnp.full((16,), L, i32) for _ in range(upad // 16))
        acc = lax.fori_loop(0, u, extract, acc0)
        for b in range(upad // 16):
            out_v[pl.ds(b * 16, 16)] = acc[b]
        pltpu.sync_copy(out_v, idx_hbm.at[wid])

    return sc_topk


def _attn_body(q_ref, k_ref, v_ref, idx_ref, out_ref, *, upad, scale, D):
    HI = jax.lax.Precision.HIGHEST
    f32 = jnp.float32
    L_Q = q_ref.shape[1]
    L_K = k_ref.shape[1]
    sub_k = jax.lax.broadcasted_iota(jnp.int32, (L_Q, upad), 0)
    lane_k = jax.lax.broadcasted_iota(jnp.int32, (upad, L_K), 1)

    for sub in range(2):
        lo = sub * D
        kh = k_ref[0, :, lo:lo + D]
        vh = v_ref[0, :, lo:lo + D]
        idxrow = idx_ref[0, pl.ds(sub, 1), :]

        pt = (sub_k == idxrow).astype(f32)
        idxcol = jnp.transpose(idxrow, (1, 0))
        pg = (lane_k == idxcol).astype(f32)
        qh = q_ref[0, :, lo:lo + D]
        qsel = jax.lax.dot_general(pg, qh, (((1,), (0,)), ((), ())),
                                   precision=HI, preferred_element_type=f32)

        ssel = jax.lax.dot_general(qsel, kh, (((1,), (1,)), ((), ())),
                                   precision=HI, preferred_element_type=f32)
        logits = ssel * scale
        p = jnp.exp(logits - jnp.max(logits, axis=1, keepdims=True))
        attn = p / jnp.sum(p, axis=1, keepdims=True)
        ctx = jax.lax.dot_general(attn, vh, (((1,), (0,)), ((), ())),
                                  precision=HI, preferred_element_type=f32)

        out_ref[0, :, lo:lo + D] = jax.lax.dot_general(
            pt, ctx, (((1,), (0,)), ((), ())),
            precision=HI, preferred_element_type=f32)


def kernel(queries, keys, values, attn_mask):
    B, L_Q, H, D = queries.shape
    L_K = keys.shape[1]
    U_part = min(_FACTOR * int(math.ceil(math.log(L_K))), L_K)
    u = min(_FACTOR * int(math.ceil(math.log(L_Q))), L_Q)
    upad = 64
    tq = 256
    bias_np, cnt_np = _sample_masks(L_Q, L_K, U_part)

    Q = queries.reshape(B, L_Q, H * D)
    K = keys.reshape(B, L_K, H * D)
    V = values.reshape(B, L_K, H * D)
    hp = H // 2
    n_pair = B * hp

    m_body = functools.partial(_m_body, tq=tq, D=D)
    m = pl.pallas_call(
        m_body,
        grid=(n_pair,),
        in_specs=[
            pl.BlockSpec((1, L_Q, 2 * D), lambda i: (i // hp, 0, i % hp)),
            pl.BlockSpec((1, L_K, 2 * D), lambda i: (i // hp, 0, i % hp)),
            pl.BlockSpec(memory_space=pl.ANY),
            pl.BlockSpec(memory_space=pl.ANY),
        ],
        out_specs=pl.BlockSpec((1, 2, L_Q), lambda i: (i, 0, 0)),
        out_shape=jax.ShapeDtypeStruct((n_pair, 2, L_Q), jnp.float32),
        scratch_shapes=[
            pltpu.VMEM((L_Q, L_K), jnp.float32),
            pltpu.VMEM((L_Q, L_K), jnp.float32),
            pltpu.SemaphoreType.DMA,
        ],
        compiler_params=pltpu.CompilerParams(
            dimension_semantics=("arbitrary",)),
    )(Q, K, jnp.asarray(bias_np), jnp.asarray(cnt_np))

    sc_topk = _make_sc_topk(B * H, L_Q, u, upad)
    idx = sc_topk(m.reshape(B * H, L_Q))
    idx3 = idx.reshape(n_pair, 2, upad)

    attn_body = functools.partial(_attn_body, upad=upad,
                                  scale=1.0 / math.sqrt(D), D=D)
    out = pl.pallas_call(
        attn_body,
        grid=(n_pair,),
        in_specs=[
            pl.BlockSpec((1, L_Q, 2 * D), lambda i: (i // hp, 0, i % hp)),
            pl.BlockSpec((1, L_K, 2 * D), lambda i: (i // hp, 0, i % hp)),
            pl.BlockSpec((1, L_K, 2 * D), lambda i: (i // hp, 0, i % hp)),
            pl.BlockSpec((1, 2, upad), lambda i: (i, 0, 0)),
        ],
        out_specs=pl.BlockSpec((1, L_Q, 2 * D), lambda i: (i // hp, 0, i % hp)),
        out_shape=jax.ShapeDtypeStruct((B, L_Q, H * D), jnp.float32),
        compiler_params=pltpu.CompilerParams(
            dimension_semantics=("arbitrary",)),
    )(Q, K, V, idx3)

    return out.reshape(B, L_Q, H, D)
